# baseline (speedup 1.0000x reference)
import functools

import jax
import jax.numpy as jnp
from jax import lax
from jax.experimental import pallas as pl
from jax.experimental.pallas import tpu as pltpu
from jax.experimental.pallas import tpu_sc as plsc

N = 10000
E = 320000
D_IN = 128
D_H = 256
HALF = 128

NC, NS = 2, 16
RPT = 624
RTAIL = N - RPT * NS
EPT = E // NS
CHUNK = 80
SUPER = 8
EPAD = 20480
NCHUNK = EPAD // CHUNK
NSUPER = NCHUNK // SUPER
EPW = E // (NC * NS)
DCHUNK = 2000
NDCHUNK = EPW // DCHUNK

BN = 1000
GRID = N // BN

_MESH = dict(core_axis_name="c", subcore_axis_name="s",
             num_cores=NC, num_subcores=NS)


def _deg_kernel(cols, w, zeros):
    mesh = plsc.VectorSubcoreMesh(**_MESH)

    @functools.partial(
        pl.kernel, mesh=mesh,
        out_type=jax.ShapeDtypeStruct((2 * N, 16), jnp.float32),
        compiler_params=pltpu.CompilerParams(use_tc_tiling_on_sc=False,
                                             needs_layout_passes=False),
        scratch_types=[
            pltpu.VMEM((DCHUNK,), jnp.int32),
            pltpu.VMEM((DCHUNK,), jnp.float32),
            pltpu.VMEM((DCHUNK, 16), jnp.float32),
            pltpu.VMEM_SHARED((N, 16), jnp.float32),
        ],
    )
    def k(cols_hbm, w_hbm, z_hbm, out_hbm, idx_v, w_v, wexp, acc):
        c = lax.axis_index("c")
        s = lax.axis_index("s")
        r0 = s * RPT
        pltpu.sync_copy(z_hbm.at[pl.ds(r0, RPT)], acc.at[pl.ds(r0, RPT)])

        @pl.when(s == 0)
        def _():
            pltpu.sync_copy(z_hbm.at[pl.ds(RPT * NS, RTAIL)],
                            acc.at[pl.ds(RPT * NS, RTAIL)])

        plsc.subcore_barrier()
        wid = s * NC + c

        def body(ki, carry):
            e0 = wid * EPW + ki * DCHUNK
            pltpu.sync_copy(cols_hbm.at[pl.ds(e0, DCHUNK)], idx_v)
            pltpu.sync_copy(w_hbm.at[pl.ds(e0, DCHUNK)], w_v)

            def expand(i, cc):
                wexp[i, :] = plsc.load_gather(
                    w_v, [jnp.full((16,), 0, jnp.int32) + i])
                return cc

            lax.fori_loop(0, DCHUNK, expand, 0)
            pltpu.sync_copy(wexp, acc.at[idx_v], add=True)
            return carry

        lax.fori_loop(0, NDCHUNK, body, 0)
        plsc.subcore_barrier()
        pltpu.sync_copy(acc.at[pl.ds(r0, RPT)],
                        out_hbm.at[pl.ds(c * N + r0, RPT)])

        @pl.when(s == 0)
        def _():
            pltpu.sync_copy(acc.at[pl.ds(RPT * NS, RTAIL)],
                            out_hbm.at[pl.ds(c * N + RPT * NS, RTAIL)])

    return k(cols, w, zeros)


def _spmm_kernel(hs_flat, rows_p, cols_p, w_p):
    mesh = plsc.VectorSubcoreMesh(**_MESH)

    @functools.partial(
        pl.kernel, mesh=mesh,
        out_type=jax.ShapeDtypeStruct((2 * N, HALF), jnp.float32),
        compiler_params=pltpu.CompilerParams(needs_layout_passes=False),
        scratch_types=[
            pltpu.VMEM((2, SUPER, CHUNK), jnp.int32),
            pltpu.VMEM((2, SUPER, CHUNK), jnp.int32),
            pltpu.VMEM((2, SUPER, CHUNK), jnp.float32),
            pltpu.VMEM((2, CHUNK, HALF), jnp.float32),
            pltpu.VMEM((2, CHUNK, HALF), jnp.float32),
            pltpu.VMEM_SHARED((N, HALF), jnp.float32),
            pltpu.SemaphoreType.DMA,
            pltpu.SemaphoreType.DMA,
            pltpu.SemaphoreType.DMA,
            pltpu.SemaphoreType.DMA,
            pltpu.SemaphoreType.DMA,
        ],
    )
    def k(hs_hbm, rows_hbm, cols_hbm, w_hbm, out_hbm,
          rows_sv, cols_sv, w_sv, mg, ms, acc,
          gsem0, gsem1, ssem0, ssem1, isem):
        gsem = (gsem0, gsem1)
        ssem = (ssem0, ssem1)
        c = lax.axis_index("c")
        s = lax.axis_index("s")
        r0 = s * RPT
        base = (c * NS + s) * NCHUNK

        def idx_copies(sm, sb):
            o = base + SUPER * sm
            return (
                pltpu.make_async_copy(rows_hbm.at[pl.ds(o, SUPER)],
                                      rows_sv.at[sb], isem),
                pltpu.make_async_copy(cols_hbm.at[pl.ds(o, SUPER)],
                                      cols_sv.at[sb], isem),
                pltpu.make_async_copy(w_hbm.at[pl.ds(o, SUPER)],
                                      w_sv.at[sb], isem),
            )

        pltpu.sync_copy(hs_hbm.at[pl.ds(c * N + r0, RPT)],
                        acc.at[pl.ds(r0, RPT)])

        @pl.when(s == 0)
        def _():
            pltpu.sync_copy(hs_hbm.at[pl.ds(c * N + RPT * NS, RTAIL)],
                            acc.at[pl.ds(RPT * NS, RTAIL)])

        plsc.subcore_barrier()

        for d in idx_copies(0, 0):
            d.start()
            d.wait()
        pltpu.async_copy(hs_hbm.at[rows_sv.at[0, 0]], mg.at[0], gsem[0])

        def chunk_body(k_, sm, sp, j):
            b = j % 2
            nb = (j + 1) % 2
            if j != SUPER - 1:
                pltpu.async_copy(hs_hbm.at[rows_sv.at[sp, j + 1]],
                                 mg.at[nb], gsem[nb])
            else:
                @pl.when(k_ + 1 < NCHUNK)
                def _():
                    for d in idx_copies(sm + 1, (sp + 1) % 2):
                        d.wait()
                    pltpu.async_copy(hs_hbm.at[rows_sv.at[(sp + 1) % 2, 0]],
                                     mg.at[nb], gsem[nb])
            pltpu.make_async_copy(hs_hbm.at[rows_sv.at[sp, j]],
                                  mg.at[b], gsem[b]).wait()
            @pl.when(k_ >= 2)
            def _():
                pltpu.make_async_copy(ms.at[b], acc.at[cols_sv.at[sp, j]],
                                      ssem[b]).wait()

            @plsc.parallel_loop(0, CHUNK, unroll=8)
            def scale(i):
                wi = plsc.load_gather(
                    w_sv, [jnp.full((16,), sp, jnp.int32),
                           jnp.full((16,), j, jnp.int32),
                           jnp.full((16,), 0, jnp.int32) + i])
                for q in range(HALF // 16):
                    ms[b, i, pl.ds(q * 16, 16)] = mg[b, i, pl.ds(q * 16, 16)] * wi

            pltpu.async_copy(ms.at[b], acc.at[cols_sv.at[sp, j]],
                             ssem[b], add=True)

        def super_pair(sm2, carry):
            for sp in (0, 1):
                sm = 2 * sm2 + sp
                @pl.when(sm + 1 < NSUPER)
                def _():
                    for d in idx_copies(sm + 1, (sp + 1) % 2):
                        d.start()
                for j in range(SUPER):
                    chunk_body(sm * SUPER + j, sm, sp, j)
            return carry

        lax.fori_loop(0, NSUPER // 2, super_pair, 0)
        for b in (0, 1):
            pltpu.make_async_copy(ms.at[b], acc.at[cols_sv.at[1, SUPER - 2 + b]],
                                  ssem[b]).wait()
        plsc.subcore_barrier()
        pltpu.sync_copy(acc.at[pl.ds(r0, RPT)],
                        out_hbm.at[pl.ds(c * N + r0, RPT)])

        @pl.when(s == 0)
        def _():
            pltpu.sync_copy(acc.at[pl.ds(RPT * NS, RTAIL)],
                            out_hbm.at[pl.ds(c * N + RPT * NS, RTAIL)])

    return k(hs_flat, rows_p, cols_p, w_p)


def _first_tc(x, W0, degt):
    def body(x_ref, w_ref, deg_ref, hs_ref, dinv_ref):
        deg = deg_ref[:, 0] + deg_ref[:, 1] + 1.0
        dinv = jnp.where(deg > 0, lax.rsqrt(jnp.maximum(deg, 1e-12)), 0.0)
        h = jnp.dot(x_ref[...].astype(jnp.bfloat16), w_ref[...].astype(jnp.bfloat16), preferred_element_type=jnp.float32)
        hs = h * dinv[:, None]
        hs_ref[0] = hs[:, :HALF]
        hs_ref[1] = hs[:, HALF:]
        dinv_ref[...] = dinv[:, None]

    return pl.pallas_call(
        body,
        grid=(GRID,),
        in_specs=[
            pl.BlockSpec((BN, D_IN), lambda i: (i, 0)),
            pl.BlockSpec((D_IN, D_H), lambda i: (0, 0)),
            pl.BlockSpec((BN, 2), lambda i: (i, 0)),
        ],
        out_specs=[
            pl.BlockSpec((2, BN, HALF), lambda i: (0, i, 0)),
            pl.BlockSpec((BN, 1), lambda i: (i, 0)),
        ],
        out_shape=[
            jax.ShapeDtypeStruct((2, N, HALF), jnp.float32),
            jax.ShapeDtypeStruct((N, 1), jnp.float32),
        ],
    )(x, W0, degt)


def _post(acc_ref, dinv_ref, b_ref, g_ref, be_ref):
    s = jnp.concatenate([acc_ref[0], acc_ref[1]], axis=1)
    out = s * dinv_ref[...] + b_ref[...][None, :]
    mu = jnp.mean(out, axis=-1, keepdims=True)
    var = jnp.mean((out - mu) ** 2, axis=-1, keepdims=True)
    z = (out - mu) * lax.rsqrt(var + 1e-5) * g_ref[...][None, :] + be_ref[...][None, :]
    return jnp.maximum(z, 0.0)


def _mid_tc(acc2, dinv, b, g, be, W):
    def body(a_ref, dinv_ref, b_ref, g_ref, be_ref, w_ref, hs_ref):
        z = _post(a_ref, dinv_ref, b_ref, g_ref, be_ref)
        h = jnp.dot(z.astype(jnp.bfloat16), w_ref[...].astype(jnp.bfloat16), preferred_element_type=jnp.float32)
        hs = h * dinv_ref[...]
        hs_ref[0] = hs[:, :HALF]
        hs_ref[1] = hs[:, HALF:]

    return pl.pallas_call(
        body,
        grid=(GRID,),
        in_specs=[
            pl.BlockSpec((2, BN, HALF), lambda i: (0, i, 0)),
            pl.BlockSpec((BN, 1), lambda i: (i, 0)),
            pl.BlockSpec((D_H,), lambda i: (0,)),
            pl.BlockSpec((D_H,), lambda i: (0,)),
            pl.BlockSpec((D_H,), lambda i: (0,)),
            pl.BlockSpec((D_H, D_H), lambda i: (0, 0)),
        ],
        out_specs=pl.BlockSpec((2, BN, HALF), lambda i: (0, i, 0)),
        out_shape=jax.ShapeDtypeStruct((2, N, HALF), jnp.float32),
    )(acc2, dinv, b, g, be, W)


def _last_tc(acc2, dinv, b, g, be, lw, lb):
    def body(a_ref, dinv_ref, b_ref, g_ref, be_ref, lw_ref, lb_ref, y_ref):
        z = _post(a_ref, dinv_ref, b_ref, g_ref, be_ref)
        y = jnp.dot(z.astype(jnp.bfloat16), lw_ref[...].astype(jnp.bfloat16), preferred_element_type=jnp.float32)
        y_ref[...] = y + lb_ref[0]

    return pl.pallas_call(
        body,
        grid=(GRID,),
        in_specs=[
            pl.BlockSpec((2, BN, HALF), lambda i: (0, i, 0)),
            pl.BlockSpec((BN, 1), lambda i: (i, 0)),
            pl.BlockSpec((D_H,), lambda i: (0,)),
            pl.BlockSpec((D_H,), lambda i: (0,)),
            pl.BlockSpec((D_H,), lambda i: (0,)),
            pl.BlockSpec((D_H, 1), lambda i: (0, 0)),
            pl.BlockSpec((1,), lambda i: (0,)),
        ],
        out_specs=pl.BlockSpec((BN, 1), lambda i: (i, 0)),
        out_shape=jax.ShapeDtypeStruct((N, 1), jnp.float32),
    )(acc2, dinv, b, g, be, lw, lb)


def kernel(x, edge_index, edge_weight, W0, b0, g0, be0, W1, b1, g1, be1,
           W2, b2, g2, be2, lw, lb):
    rows = edge_index[0].astype(jnp.int32)
    cols = edge_index[1].astype(jnp.int32)
    w = edge_weight.astype(jnp.float32)
    pad = ((0, 0), (0, EPAD - EPT))
    r3 = jnp.pad(rows.reshape(NS, EPT), pad)
    c3 = jnp.pad(cols.reshape(NS, EPT), pad)
    w3 = jnp.pad(w.reshape(NS, EPT), pad)
    rows_p = jnp.concatenate([r3, r3 + N], 0).reshape(2 * NS * NCHUNK, CHUNK)
    cols_p = jnp.concatenate([c3, c3], 0).reshape(2 * NS * NCHUNK, CHUNK)
    w_p = jnp.concatenate([w3, w3], 0).reshape(2 * NS * NCHUNK, CHUNK)
    zeros = jnp.zeros((N, 16), jnp.float32)

    degp = _deg_kernel(cols, w, zeros)
    degt = jnp.transpose(degp[:, 0].reshape(2, N), (1, 0))

    hs2, dinv = _first_tc(x, W0, degt)
    for (b, g, be, Wn) in ((b0, g0, be0, W1), (b1, g1, be1, W2)):
        acc = _spmm_kernel(hs2.reshape(2 * N, HALF), rows_p, cols_p, w_p)
        hs2 = _mid_tc(acc.reshape(2, N, HALF), dinv, b, g, be, Wn)
    acc = _spmm_kernel(hs2.reshape(2 * N, HALF), rows_p, cols_p, w_p)
    y = _last_tc(acc.reshape(2, N, HALF), dinv, b2, g2, be2, lw, lb)
    return y.reshape(N)


# speedup vs baseline: 1.0170x; 1.0170x over previous
import functools

import jax
import jax.numpy as jnp
from jax import lax
from jax.experimental import pallas as pl
from jax.experimental.pallas import tpu as pltpu
from jax.experimental.pallas import tpu_sc as plsc

N = 10000
E = 320000
D_IN = 128
D_H = 256
HALF = 128

NC, NS = 2, 16
RPT = 624
RTAIL = N - RPT * NS
EPT = E // NS
CHUNK = 200
NCHUNK = EPT // CHUNK
EPW = E // (NC * NS)
DCHUNK = 2000
NDCHUNK = EPW // DCHUNK

BN = 1000
GRID = N // BN

_MESH = dict(core_axis_name="c", subcore_axis_name="s",
             num_cores=NC, num_subcores=NS)


def _deg_kernel(cols, w, zeros):
    mesh = plsc.VectorSubcoreMesh(**_MESH)

    @functools.partial(
        pl.kernel, mesh=mesh,
        out_type=jax.ShapeDtypeStruct((2 * N, 16), jnp.float32),
        compiler_params=pltpu.CompilerParams(use_tc_tiling_on_sc=False,
                                             needs_layout_passes=False),
        scratch_types=[
            pltpu.VMEM((DCHUNK,), jnp.int32),
            pltpu.VMEM((DCHUNK,), jnp.float32),
            pltpu.VMEM((DCHUNK, 16), jnp.float32),
            pltpu.VMEM_SHARED((N, 16), jnp.float32),
        ],
    )
    def k(cols_hbm, w_hbm, z_hbm, out_hbm, idx_v, w_v, wexp, acc):
        c = lax.axis_index("c")
        s = lax.axis_index("s")
        r0 = s * RPT
        pltpu.sync_copy(z_hbm.at[pl.ds(r0, RPT)], acc.at[pl.ds(r0, RPT)])

        @pl.when(s == 0)
        def _():
            pltpu.sync_copy(z_hbm.at[pl.ds(RPT * NS, RTAIL)],
                            acc.at[pl.ds(RPT * NS, RTAIL)])

        plsc.subcore_barrier()
        wid = s * NC + c

        def body(ki, carry):
            e0 = wid * EPW + ki * DCHUNK
            pltpu.sync_copy(cols_hbm.at[pl.ds(e0, DCHUNK)], idx_v)
            pltpu.sync_copy(w_hbm.at[pl.ds(e0, DCHUNK)], w_v)

            def expand(i, cc):
                wexp[i, :] = plsc.load_gather(
                    w_v, [jnp.full((16,), 0, jnp.int32) + i])
                return cc

            lax.fori_loop(0, DCHUNK, expand, 0)
            pltpu.sync_copy(wexp, acc.at[idx_v], add=True)
            return carry

        lax.fori_loop(0, NDCHUNK, body, 0)
        plsc.subcore_barrier()
        pltpu.sync_copy(acc.at[pl.ds(r0, RPT)],
                        out_hbm.at[pl.ds(c * N + r0, RPT)])

        @pl.when(s == 0)
        def _():
            pltpu.sync_copy(acc.at[pl.ds(RPT * NS, RTAIL)],
                            out_hbm.at[pl.ds(c * N + RPT * NS, RTAIL)])

    return k(cols, w, zeros)


def _spmm_kernel(hs_flat, rows2, cols, w):
    mesh = plsc.VectorSubcoreMesh(**_MESH)

    @functools.partial(
        pl.kernel, mesh=mesh,
        out_type=jax.ShapeDtypeStruct((2 * N, HALF), jnp.float32),
        compiler_params=pltpu.CompilerParams(needs_layout_passes=False),
        scratch_types=[
            pltpu.VMEM((CHUNK,), jnp.int32),
            pltpu.VMEM((CHUNK,), jnp.int32),
            pltpu.VMEM((CHUNK,), jnp.float32),
            pltpu.VMEM((CHUNK, HALF), jnp.float32),
            pltpu.VMEM_SHARED((N, HALF), jnp.float32),
            pltpu.SemaphoreType.DMA,
        ],
    )
    def k(hs_hbm, rows_hbm, cols_hbm, w_hbm, out_hbm,
          rows_v, cols_v, w_v, msgs, acc, sem):
        c = lax.axis_index("c")
        s = lax.axis_index("s")
        r0 = s * RPT
        pltpu.sync_copy(hs_hbm.at[pl.ds(c * N + r0, RPT)],
                        acc.at[pl.ds(r0, RPT)])

        @pl.when(s == 0)
        def _():
            pltpu.sync_copy(hs_hbm.at[pl.ds(c * N + RPT * NS, RTAIL)],
                            acc.at[pl.ds(RPT * NS, RTAIL)])

        plsc.subcore_barrier()

        def body(ki, carry):
            e0 = c * E + s * EPT + ki * CHUNK
            ec = s * EPT + ki * CHUNK
            pltpu.sync_copy(rows_hbm.at[pl.ds(e0, CHUNK)], rows_v)
            pltpu.sync_copy(cols_hbm.at[pl.ds(ec, CHUNK)], cols_v)
            pltpu.sync_copy(w_hbm.at[pl.ds(ec, CHUNK)], w_v)
            pltpu.async_copy(hs_hbm.at[rows_v], msgs, sem).wait()

            @plsc.parallel_loop(0, CHUNK, unroll=8)
            def scale(i):
                wi = plsc.load_gather(w_v, [jnp.full((16,), 0, jnp.int32) + i])
                for j in range(HALF // 16):
                    msgs[i, pl.ds(j * 16, 16)] = msgs[i, pl.ds(j * 16, 16)] * wi
            pltpu.sync_copy(msgs, acc.at[cols_v], add=True)
            return carry

        lax.fori_loop(0, NCHUNK, body, 0)
        plsc.subcore_barrier()
        pltpu.sync_copy(acc.at[pl.ds(r0, RPT)],
                        out_hbm.at[pl.ds(c * N + r0, RPT)])

        @pl.when(s == 0)
        def _():
            pltpu.sync_copy(acc.at[pl.ds(RPT * NS, RTAIL)],
                            out_hbm.at[pl.ds(c * N + RPT * NS, RTAIL)])

    return k(hs_flat, rows2, cols, w)


def _first_tc(x, W0, degt):
    def body(x_ref, w_ref, deg_ref, hs_ref, dinv_ref):
        deg = deg_ref[:, 0] + deg_ref[:, 1] + 1.0
        dinv = jnp.where(deg > 0, lax.rsqrt(jnp.maximum(deg, 1e-12)), 0.0)
        h = jnp.dot(x_ref[...].astype(jnp.bfloat16), w_ref[...].astype(jnp.bfloat16), preferred_element_type=jnp.float32)
        hs = h * dinv[:, None]
        hs_ref[0] = hs[:, :HALF]
        hs_ref[1] = hs[:, HALF:]
        dinv_ref[...] = dinv[:, None]

    return pl.pallas_call(
        body,
        grid=(GRID,),
        in_specs=[
            pl.BlockSpec((BN, D_IN), lambda i: (i, 0)),
            pl.BlockSpec((D_IN, D_H), lambda i: (0, 0)),
            pl.BlockSpec((BN, 2), lambda i: (i, 0)),
        ],
        out_specs=[
            pl.BlockSpec((2, BN, HALF), lambda i: (0, i, 0)),
            pl.BlockSpec((BN, 1), lambda i: (i, 0)),
        ],
        out_shape=[
            jax.ShapeDtypeStruct((2, N, HALF), jnp.float32),
            jax.ShapeDtypeStruct((N, 1), jnp.float32),
        ],
    )(x, W0, degt)


def _post(acc_ref, dinv_ref, b_ref, g_ref, be_ref):
    s = jnp.concatenate([acc_ref[0], acc_ref[1]], axis=1)
    out = s * dinv_ref[...] + b_ref[...][None, :]
    mu = jnp.mean(out, axis=-1, keepdims=True)
    var = jnp.mean((out - mu) ** 2, axis=-1, keepdims=True)
    z = (out - mu) * lax.rsqrt(var + 1e-5) * g_ref[...][None, :] + be_ref[...][None, :]
    return jnp.maximum(z, 0.0)


def _mid_tc(acc2, dinv, b, g, be, W):
    def body(a_ref, dinv_ref, b_ref, g_ref, be_ref, w_ref, hs_ref):
        z = _post(a_ref, dinv_ref, b_ref, g_ref, be_ref)
        h = jnp.dot(z.astype(jnp.bfloat16), w_ref[...].astype(jnp.bfloat16), preferred_element_type=jnp.float32)
        hs = h * dinv_ref[...]
        hs_ref[0] = hs[:, :HALF]
        hs_ref[1] = hs[:, HALF:]

    return pl.pallas_call(
        body,
        grid=(GRID,),
        in_specs=[
            pl.BlockSpec((2, BN, HALF), lambda i: (0, i, 0)),
            pl.BlockSpec((BN, 1), lambda i: (i, 0)),
            pl.BlockSpec((D_H,), lambda i: (0,)),
            pl.BlockSpec((D_H,), lambda i: (0,)),
            pl.BlockSpec((D_H,), lambda i: (0,)),
            pl.BlockSpec((D_H, D_H), lambda i: (0, 0)),
        ],
        out_specs=pl.BlockSpec((2, BN, HALF), lambda i: (0, i, 0)),
        out_shape=jax.ShapeDtypeStruct((2, N, HALF), jnp.float32),
    )(acc2, dinv, b, g, be, W)


def _last_tc(acc2, dinv, b, g, be, lw, lb):
    def body(a_ref, dinv_ref, b_ref, g_ref, be_ref, lw_ref, lb_ref, y_ref):
        z = _post(a_ref, dinv_ref, b_ref, g_ref, be_ref)
        y = jnp.dot(z.astype(jnp.bfloat16), lw_ref[...].astype(jnp.bfloat16), preferred_element_type=jnp.float32)
        y_ref[...] = y + lb_ref[0]

    return pl.pallas_call(
        body,
        grid=(GRID,),
        in_specs=[
            pl.BlockSpec((2, BN, HALF), lambda i: (0, i, 0)),
            pl.BlockSpec((BN, 1), lambda i: (i, 0)),
            pl.BlockSpec((D_H,), lambda i: (0,)),
            pl.BlockSpec((D_H,), lambda i: (0,)),
            pl.BlockSpec((D_H,), lambda i: (0,)),
            pl.BlockSpec((D_H, 1), lambda i: (0, 0)),
            pl.BlockSpec((1,), lambda i: (0,)),
        ],
        out_specs=pl.BlockSpec((BN, 1), lambda i: (i, 0)),
        out_shape=jax.ShapeDtypeStruct((N, 1), jnp.float32),
    )(acc2, dinv, b, g, be, lw, lb)


def kernel(x, edge_index, edge_weight, W0, b0, g0, be0, W1, b1, g1, be1,
           W2, b2, g2, be2, lw, lb):
    rows = edge_index[0].astype(jnp.int32)
    cols = edge_index[1].astype(jnp.int32)
    w = edge_weight.astype(jnp.float32)
    rows2 = jnp.concatenate([rows, rows + N])
    zeros = jnp.zeros((N, 16), jnp.float32)

    degp = _deg_kernel(cols, w, zeros)
    degt = jnp.transpose(degp[:, 0].reshape(2, N), (1, 0))

    hs2, dinv = _first_tc(x, W0, degt)
    for (b, g, be, Wn) in ((b0, g0, be0, W1), (b1, g1, be1, W2)):
        acc = _spmm_kernel(hs2.reshape(2 * N, HALF), rows2, cols, w)
        hs2 = _mid_tc(acc.reshape(2, N, HALF), dinv, b, g, be, Wn)
    acc = _spmm_kernel(hs2.reshape(2 * N, HALF), rows2, cols, w)
    y = _last_tc(acc.reshape(2, N, HALF), dinv, b2, g2, be2, lw, lb)
    return y.reshape(N)
